# baseline (speedup 1.0000x reference)
import functools

import jax
import jax.numpy as jnp
from jax import lax
from jax.experimental import pallas as pl
from jax.experimental.pallas import tpu as pltpu
from jax.experimental.pallas import tpu_sc as plsc

T = 8192
E = 8
D = 1024
C = 1024
L = 16
NC = 2
NS = 16
TPT = T // NS
GROUPS = T // L
E_PER_CORE = E // NC
OWN = C // NS
CH = 32
CSH = 5
DBLK = D // L
NRING = 1024
NRROW = NRING // CH

_mesh = plsc.VectorSubcoreMesh(core_axis_name="c", subcore_axis_name="s")

_DNUMS = lax.GatherDimensionNumbers(
    offset_dims=(), collapsed_slice_dims=(0,), start_index_map=(0,))


def _splat(vec, r):
    idx = jnp.full((L, 1), r, jnp.int32)
    return lax.gather(vec, idx, _DNUMS, (1,),
                      mode=lax.GatherScatterMode.PROMISE_IN_BOUNDS)


@functools.partial(
    pl.kernel,
    out_type=jax.ShapeDtypeStruct((E, C, D), jnp.float32),
    mesh=_mesh,
    scratch_types=[
        pltpu.VMEM((T,), jnp.float32),
        pltpu.VMEM((T,), jnp.float32),
        pltpu.VMEM((T,), jnp.int32),
        pltpu.VMEM((NRROW + 1, CH), jnp.int32),
        pltpu.VMEM((CH, D), jnp.float32),
        pltpu.VMEM((OWN, D), jnp.float32),
        pltpu.VMEM_SHARED((T,), jnp.float32),
        pltpu.SemaphoreType.DMA,
    ],
    compiler_params=pltpu.CompilerParams(needs_layout_passes=False),
)
def _dispatch(in_hbm, route_hbm, score_hbm, out_hbm,
              thr_v, score_col, route_col,
              selid_v, rows_a, acc_v,
              thr_sh, sem_a):
    c = lax.axis_index("c")
    s = lax.axis_index("s")
    tok0 = s * TPT

    lanes = lax.iota(jnp.int32, L)
    neg = jnp.full((L,), -jnp.inf, jnp.float32)
    zf = jnp.zeros((L,), jnp.float32)
    zi = jnp.zeros((L,), jnp.int32)

    def _pre(i, cc):
        for q in range(CH // L):
            selid_v[i, pl.ds(q * L, L)] = zi
        return cc
    lax.fori_loop(0, NRROW + 1, _pre, 0)

    for e in range(E):
        pltpu.sync_copy(score_hbm.at[e, pl.ds(tok0, TPT)],
                        score_col.at[pl.ds(e * TPT, TPT)])

    def _thr(g, cc):
        sc = [score_col[pl.ds(e * TPT + g * L, L)] for e in range(E)]
        m1 = sc[0]
        for e in range(1, E):
            m1 = jnp.maximum(m1, sc[e])
        excl = jnp.zeros((L,), jnp.bool_)
        m2 = neg
        for e in range(E):
            is_first = (sc[e] == m1) & (~excl)
            m2 = jnp.maximum(m2, jnp.where(is_first, neg, sc[e]))
            excl = excl | is_first
        thr_v[pl.ds(tok0 + g * L, L)] = m2
        return cc
    lax.fori_loop(0, TPT // L, _thr, 0)

    pltpu.sync_copy(thr_v.at[pl.ds(tok0, TPT)], thr_sh.at[pl.ds(tok0, TPT)])
    plsc.subcore_barrier()
    pltpu.sync_copy(thr_sh, thr_v)

    slot_base = s * OWN

    def _acc_chunk(off, cntv, slot_base):
        jr = lax.shift_right_logical(off, CSH)
        for q in range(CH // L):
            idv = selid_v[jr, pl.ds(q * L, L)]
            sce = plsc.load_gather(score_col, [idv])
            th = plsc.load_gather(thr_v, [idv])
            slotf = plsc.load_gather(route_col, [idv]) - slot_base
            use = (((off + q * L + lanes) < cntv) & (sce >= th)
                   & (sce > 0.0) & (slotf >= 0)
                   & (slotf < jnp.full((L,), OWN, jnp.int32)))
            gate = jnp.where(use, sce, zf)
            slotv = jnp.where(use, slotf, zi)

            def _acc(r, cc):
                g = _splat(gate, r)
                sl = _splat(slotv, r)
                for k in range(DBLK):
                    contrib = rows_a[q * L + r, pl.ds(k * L, L)] * g
                    plsc.addupdate_scatter(acc_v, [sl, k * L + lanes],
                                           contrib)
                return cc
            lax.fori_loop(0, L, _acc, 0)

    def _phase(p, cc):
        expert = c * E_PER_CORE + p
        pltpu.sync_copy(score_hbm.at[expert], score_col)
        pltpu.sync_copy(route_hbm.at[expert], route_col)

        def _zero(r, cc):
            for k in range(DBLK):
                acc_v[r, pl.ds(k * L, L)] = zf
            return cc
        lax.fori_loop(0, OWN, _zero, 0)

        def _scan(g, cntv):
            sce = score_col[pl.ds(g * L, L)]
            th = thr_v[pl.ds(g * L, L)]
            slot = route_col[pl.ds(g * L, L)] - slot_base
            msk = ((sce >= th) & (sce > 0.0) & (slot >= 0)
                   & (slot < jnp.full((L,), OWN, jnp.int32)))
            inc = msk.astype(jnp.int32)
            cs = plsc.cumsum(inc)
            raw = cntv + cs - 1
            pos = jnp.where(msk & (raw < jnp.full((L,), NRING, jnp.int32)),
                            raw, NRING + lanes)
            phi = lax.shift_right_logical(pos, CSH)
            plo = pos & (CH - 1)
            plsc.store_scatter(selid_v, [phi, plo], g * L + lanes)
            return cntv + _splat(cs, L - 1)
        cntv = lax.fori_loop(0, GROUPS, _scan, zi)
        cnt = jnp.minimum(jnp.max(cntv), NRING)

        cntv16 = jnp.full((L,), cnt, jnp.int32)
        n = lax.shift_right_logical(cnt + CH - 1, CSH)

        def _fin(j, cc):
            m = j * CH
            jr = lax.shift_right_logical(m, CSH)
            pltpu.async_copy(in_hbm.at[selid_v.at[jr]], rows_a, sem_a).wait()
            _acc_chunk(m, cntv16, slot_base)
            return cc
        lax.fori_loop(0, n, _fin, 0)

        pltpu.sync_copy(acc_v, out_hbm.at[expert, pl.ds(slot_base, OWN)])
        return cc

    lax.fori_loop(0, E_PER_CORE, _phase, 0)


def kernel(in_flow, route_indices, loads, capacities, score):
    del loads, capacities
    return _dispatch(in_flow, route_indices.T, score.T)


# speedup vs baseline: 1.0855x; 1.0855x over previous
import functools

import jax
import jax.numpy as jnp
from jax import lax
from jax.experimental import pallas as pl
from jax.experimental.pallas import tpu as pltpu
from jax.experimental.pallas import tpu_sc as plsc

T = 8192
E = 8
D = 1024
C = 1024
L = 16
NC = 2
NS = 16
TPT = T // NS
GROUPS = T // L
E_PER_CORE = E // NC
OWN = C // NS
CH = 16
CSH = 4
DBLK = D // L
NRING = 1024
NRROW = NRING // CH

_mesh = plsc.VectorSubcoreMesh(core_axis_name="c", subcore_axis_name="s")

_DNUMS = lax.GatherDimensionNumbers(
    offset_dims=(), collapsed_slice_dims=(0,), start_index_map=(0,))


def _splat(vec, r):
    idx = jnp.full((L, 1), r, jnp.int32)
    return lax.gather(vec, idx, _DNUMS, (1,),
                      mode=lax.GatherScatterMode.PROMISE_IN_BOUNDS)


@functools.partial(
    pl.kernel,
    out_type=jax.ShapeDtypeStruct((E, C, D), jnp.float32),
    mesh=_mesh,
    scratch_types=[
        pltpu.VMEM((T,), jnp.float32),
        pltpu.VMEM((T,), jnp.float32),
        pltpu.VMEM((T,), jnp.int32),
        pltpu.VMEM((NRROW + 1, CH), jnp.int32),
        pltpu.VMEM((CH, D), jnp.float32),
        pltpu.VMEM((OWN, D), jnp.float32),
        pltpu.VMEM_SHARED((T,), jnp.float32),
        pltpu.SemaphoreType.DMA,
    ],
    compiler_params=pltpu.CompilerParams(needs_layout_passes=False),
)
def _dispatch(in_hbm, route_hbm, score_hbm, out_hbm,
              thr_v, score_col, route_col,
              selid_v, rows_a, acc_v,
              thr_sh, sem_a):
    c = lax.axis_index("c")
    s = lax.axis_index("s")
    tok0 = s * TPT

    lanes = lax.iota(jnp.int32, L)
    neg = jnp.full((L,), -jnp.inf, jnp.float32)
    zf = jnp.zeros((L,), jnp.float32)
    zi = jnp.zeros((L,), jnp.int32)

    def _pre(i, cc):
        for q in range(CH // L):
            selid_v[i, pl.ds(q * L, L)] = zi
        return cc
    lax.fori_loop(0, NRROW + 1, _pre, 0)

    for e in range(E):
        pltpu.sync_copy(score_hbm.at[e, pl.ds(tok0, TPT)],
                        score_col.at[pl.ds(e * TPT, TPT)])

    def _thr(g, cc):
        sc = [score_col[pl.ds(e * TPT + g * L, L)] for e in range(E)]
        m1 = sc[0]
        for e in range(1, E):
            m1 = jnp.maximum(m1, sc[e])
        excl = jnp.zeros((L,), jnp.bool_)
        m2 = neg
        for e in range(E):
            is_first = (sc[e] == m1) & (~excl)
            m2 = jnp.maximum(m2, jnp.where(is_first, neg, sc[e]))
            excl = excl | is_first
        thr_v[pl.ds(tok0 + g * L, L)] = m2
        return cc
    lax.fori_loop(0, TPT // L, _thr, 0)

    pltpu.sync_copy(thr_v.at[pl.ds(tok0, TPT)], thr_sh.at[pl.ds(tok0, TPT)])
    plsc.subcore_barrier()
    pltpu.sync_copy(thr_sh, thr_v)

    slot_base = s * OWN

    def _acc_chunk(off, cntv, slot_base):
        jr = lax.shift_right_logical(off, CSH)
        idv = selid_v[jr, :]
        sce = plsc.load_gather(score_col, [idv])
        th = plsc.load_gather(thr_v, [idv])
        slotf = plsc.load_gather(route_col, [idv]) - slot_base
        use = (((off + lanes) < cntv) & (sce >= th) & (sce > 0.0)
               & (slotf >= 0) & (slotf < jnp.full((L,), OWN, jnp.int32)))
        gate = jnp.where(use, sce, zf)
        slotv = jnp.where(use, slotf, zi)

        def _acc(r, cc):
            g = _splat(gate, r)
            sl = _splat(slotv, r)
            for k in range(DBLK):
                contrib = rows_a[r, pl.ds(k * L, L)] * g
                plsc.addupdate_scatter(acc_v, [sl, k * L + lanes],
                                       contrib)
            return cc
        lax.fori_loop(0, L, _acc, 0)

    def _phase(p, cc):
        expert = c * E_PER_CORE + p
        pltpu.sync_copy(score_hbm.at[expert], score_col)
        pltpu.sync_copy(route_hbm.at[expert], route_col)

        def _zero(r, cc):
            for k in range(DBLK):
                acc_v[r, pl.ds(k * L, L)] = zf
            return cc
        lax.fori_loop(0, OWN, _zero, 0)

        def _scan(g, cntv):
            sce = score_col[pl.ds(g * L, L)]
            th = thr_v[pl.ds(g * L, L)]
            slot = route_col[pl.ds(g * L, L)] - slot_base
            msk = ((sce >= th) & (sce > 0.0) & (slot >= 0)
                   & (slot < jnp.full((L,), OWN, jnp.int32)))
            inc = msk.astype(jnp.int32)
            cs = plsc.cumsum(inc)
            raw = cntv + cs - 1
            pos = jnp.where(msk & (raw < jnp.full((L,), NRING, jnp.int32)),
                            raw, NRING + lanes)
            phi = lax.shift_right_logical(pos, CSH)
            plo = pos & (CH - 1)
            plsc.store_scatter(selid_v, [phi, plo], g * L + lanes)
            return cntv + _splat(cs, L - 1)
        cntv = lax.fori_loop(0, GROUPS, _scan, zi)
        cnt = jnp.minimum(jnp.max(cntv), NRING)

        cntv16 = jnp.full((L,), cnt, jnp.int32)
        n = lax.shift_right_logical(cnt + CH - 1, CSH)

        def _fin(j, cc):
            m = j * CH
            jr = lax.shift_right_logical(m, CSH)
            pltpu.async_copy(in_hbm.at[selid_v.at[jr]], rows_a, sem_a).wait()
            _acc_chunk(m, cntv16, slot_base)
            return cc
        lax.fori_loop(0, n, _fin, 0)

        pltpu.sync_copy(acc_v, out_hbm.at[expert, pl.ds(slot_base, OWN)])
        return cc

    lax.fori_loop(0, E_PER_CORE, _phase, 0)


def kernel(in_flow, route_indices, loads, capacities, score):
    del loads, capacities
    return _dispatch(in_flow, route_indices.T, score.T)
